# Initial kernel scaffold; baseline (speedup 1.0000x reference)
import jax
import jax.numpy as jnp

B = 64
H = 128
M = 128
STEPS = 32
N = B * 81
E = B * 1620


def _ln(x, g, b):
    mu = jnp.mean(x, axis=-1, keepdims=True)
    var = jnp.mean((x - mu) ** 2, axis=-1, keepdims=True)
    return (x - mu) / jnp.sqrt(var + 1e-5) * g + b


def kernel(x, edge_index, W_in, b_in, ln_in_g, ln_in_b, pos, W_m1, b_m1, W_m2, b_m2, W_ih, b_ih, W_hh, b_hh, ln_g, ln_b, W_out, b_out):
    h = _ln(x @ W_in + b_in, ln_in_g, ln_in_b) + pos[None]
    h = h.reshape(N, H)
    src = edge_index[0]
    dst = edge_index[1]
    W1a = W_m1[:H]
    W1b = W_m1[H:]
    deg = jnp.zeros((N,), jnp.float32).at[dst].add(1.0)
    outs = []
    for _ in range(STEPS):
        a = h @ W1a + b_m1
        bb = h @ W1b
        r_e = jnp.maximum(a[src] + bb[dst], 0.0)
        rsum = jnp.zeros((N, M), jnp.float32).at[dst].add(r_e)
        agg = rsum @ W_m2 + deg[:, None] * b_m2
        gi = agg @ W_ih.T + b_ih
        gh = h @ W_hh.T + b_hh
        i_r, i_z, i_n = jnp.split(gi, 3, axis=-1)
        h_r, h_z, h_n = jnp.split(gh, 3, axis=-1)
        r = jax.nn.sigmoid(i_r + h_r)
        z = jax.nn.sigmoid(i_z + h_z)
        n = jnp.tanh(i_n + r * h_n)
        h = (1.0 - z) * n + z * h
        h = _ln(h, ln_g, ln_b)
        outs.append((h @ W_out + b_out).reshape(B, 81, 9))
    return jnp.stack(outs)


# speedup vs baseline: 4.3550x; 4.3550x over previous
import functools
import jax
import jax.numpy as jnp
from jax import lax
from jax.experimental import pallas as pl
from jax.experimental.pallas import tpu as pltpu
from jax.experimental.pallas import tpu_sc as plsc

B = 64
H = 128
STEPS = 32
N = B * 81
E = B * 1620

NC = 2
NS = 16
NW = NC * NS
EPW = E // NW
K = 120
NCHUNK = EPW // K
RPT = 328
NPAD = RPT * NS

BLK = 648
GRID = N // BLK


def _zero_rows(ref, nrows, width):
    zero16 = jnp.zeros((16,), jnp.float32)

    def zb(i, carry):
        for j in range(width // 16):
            ref[i, pl.ds(j * 16, 16)] = zero16
        return carry
    lax.fori_loop(0, nrows, zb, 0)


def _sc_edge_body(a_hbm, c_hbm, src_hbm, dst_hbm, r_out,
                  si_v, di_v, a_v, c_v, z_v, acc_sh, sem_a, sem_c):
    core = lax.axis_index("c")
    sid = lax.axis_index("s")

    _zero_rows(z_v, RPT, H)

    base_r = sid * RPT
    pltpu.sync_copy(z_v, acc_sh.at[pl.ds(base_r, RPT)])
    plsc.subcore_barrier()

    wid = core * NS + sid
    ebase = wid * EPW

    def chunk(j, carry):
        b = ebase + j * K
        pltpu.sync_copy(src_hbm.at[pl.ds(b, K)], si_v)
        pltpu.sync_copy(dst_hbm.at[pl.ds(b, K)], di_v)
        cp_a = pltpu.async_copy(a_hbm.at[si_v], a_v, sem_a)
        cp_c = pltpu.async_copy(c_hbm.at[di_v], c_v, sem_c)
        cp_a.wait()
        cp_c.wait()

        def row(k, inner):
            for jj in range(H // 16):
                s = pl.ds(jj * 16, 16)
                a_v[k, s] = jnp.maximum(a_v[k, s] + c_v[k, s], 0.0)
            return inner
        lax.fori_loop(0, K, row, 0)

        pltpu.sync_copy(a_v, acc_sh.at[di_v], add=True)
        return carry
    lax.fori_loop(0, NCHUNK, chunk, 0)

    plsc.subcore_barrier()
    pltpu.sync_copy(acc_sh.at[pl.ds(base_r, RPT)],
                    r_out.at[core, pl.ds(base_r, RPT)])


def _make_sc_edge():
    mesh = plsc.VectorSubcoreMesh(core_axis_name="c", subcore_axis_name="s")
    return functools.partial(
        pl.kernel, mesh=mesh,
        out_type=jax.ShapeDtypeStruct((NC, NPAD, H), jnp.float32),
        scratch_types=[
            pltpu.VMEM((K,), jnp.int32),
            pltpu.VMEM((K,), jnp.int32),
            pltpu.VMEM((K, H), jnp.float32),
            pltpu.VMEM((K, H), jnp.float32),
            pltpu.VMEM((RPT, H), jnp.float32),
            pltpu.VMEM_SHARED((NPAD, H), jnp.float32),
            pltpu.SemaphoreType.DMA,
            pltpu.SemaphoreType.DMA,
        ],
    )(_sc_edge_body)


def _sc_cnt_body(dst_hbm, c_out, di_v, one_v, z_v, acc_sh):
    core = lax.axis_index("c")
    sid = lax.axis_index("s")

    _zero_rows(z_v, RPT, H)
    _zero_rows(one_v, K, H)
    lane = lax.iota(jnp.int32, 16)
    onev = jnp.where(lane == 0, 1.0, 0.0).astype(jnp.float32)

    def so(i, carry):
        one_v[i, pl.ds(0, 16)] = onev
        return carry
    lax.fori_loop(0, K, so, 0)

    base_r = sid * RPT
    pltpu.sync_copy(z_v, acc_sh.at[pl.ds(base_r, RPT)])
    plsc.subcore_barrier()

    wid = core * NS + sid
    ebase = wid * EPW

    def chunk(j, carry):
        b = ebase + j * K
        pltpu.sync_copy(dst_hbm.at[pl.ds(b, K)], di_v)
        pltpu.sync_copy(one_v, acc_sh.at[di_v], add=True)
        return carry
    lax.fori_loop(0, NCHUNK, chunk, 0)

    plsc.subcore_barrier()
    pltpu.sync_copy(acc_sh.at[pl.ds(base_r, RPT)],
                    c_out.at[core, pl.ds(base_r, RPT)])


def _make_sc_cnt():
    mesh = plsc.VectorSubcoreMesh(core_axis_name="c", subcore_axis_name="s")
    return functools.partial(
        pl.kernel, mesh=mesh,
        out_type=jax.ShapeDtypeStruct((NC, NPAD, H), jnp.float32),
        scratch_types=[
            pltpu.VMEM((K,), jnp.int32),
            pltpu.VMEM((K, H), jnp.float32),
            pltpu.VMEM((RPT, H), jnp.float32),
            pltpu.VMEM_SHARED((NPAD, H), jnp.float32),
        ],
    )(_sc_cnt_body)


def _wcomb_body(wm2_ref, wiht_ref, bm2_ref, out_ref, brow_ref):
    out_ref[...] = jnp.dot(wm2_ref[...], wiht_ref[...],
                           preferred_element_type=jnp.float32)
    brow_ref[...] = jnp.dot(bm2_ref[...], wiht_ref[...],
                            preferred_element_type=jnp.float32)


def _wcomb(wm2, wiht, bm2):
    return pl.pallas_call(
        _wcomb_body,
        out_shape=[
            jax.ShapeDtypeStruct((H, 3 * H), jnp.float32),
            jax.ShapeDtypeStruct((1, 3 * H), jnp.float32),
        ],
    )(wm2, wiht, bm2)


def _bias_body(c0_ref, c1_ref, brow_ref, bih_ref, out_ref):
    cnt = c0_ref[:, :1] + c1_ref[:, :1]
    out_ref[...] = cnt * brow_ref[...] + bih_ref[...]


def _bias(c0, c1, brow, bih):
    full = lambda i: (0, 0)
    rows = lambda i: (i, 0)
    return pl.pallas_call(
        _bias_body,
        grid=(GRID,),
        in_specs=[
            pl.BlockSpec((BLK, H), rows),
            pl.BlockSpec((BLK, H), rows),
            pl.BlockSpec((1, 3 * H), full),
            pl.BlockSpec((1, 3 * H), full),
        ],
        out_specs=pl.BlockSpec((BLK, 3 * H), rows),
        out_shape=jax.ShapeDtypeStruct((N, 3 * H), jnp.float32),
    )(c0, c1, brow, bih)


def _init_body(x_ref, win_ref, bin_ref, g_ref, b_ref, pos_ref,
               wa_ref, wc_ref, bm1_ref,
               h_ref, a_ref, c_ref):
    t = jnp.dot(x_ref[...], win_ref[...], preferred_element_type=jnp.float32)
    t = t + bin_ref[...]
    mu = jnp.mean(t, axis=-1, keepdims=True)
    var = jnp.mean((t - mu) ** 2, axis=-1, keepdims=True)
    h = (t - mu) / jnp.sqrt(var + 1e-5) * g_ref[...] + b_ref[...]
    h = h + pos_ref[...]
    h_ref[...] = h
    a_ref[...] = jnp.dot(h, wa_ref[...], preferred_element_type=jnp.float32) + bm1_ref[...]
    c_ref[...] = jnp.dot(h, wc_ref[...], preferred_element_type=jnp.float32)


def _init(x16, win16, b_in, g_in, bb_in, posf, wa, wc, bm1):
    full = lambda i: (0, 0)
    rows = lambda i: (i, 0)
    return pl.pallas_call(
        _init_body,
        grid=(GRID,),
        in_specs=[
            pl.BlockSpec((BLK, 16), rows),
            pl.BlockSpec((16, H), full),
            pl.BlockSpec((1, H), full),
            pl.BlockSpec((1, H), full),
            pl.BlockSpec((1, H), full),
            pl.BlockSpec((BLK, H), rows),
            pl.BlockSpec((H, H), full),
            pl.BlockSpec((H, H), full),
            pl.BlockSpec((1, H), full),
        ],
        out_specs=[
            pl.BlockSpec((BLK, H), rows),
            pl.BlockSpec((BLK, H), rows),
            pl.BlockSpec((BLK, H), rows),
        ],
        out_shape=[
            jax.ShapeDtypeStruct((N, H), jnp.float32),
            jax.ShapeDtypeStruct((N, H), jnp.float32),
            jax.ShapeDtypeStruct((N, H), jnp.float32),
        ],
    )(x16, win16, b_in, g_in, bb_in, posf, wa, wc, bm1)


def _step_body(r0_ref, r1_ref, h_ref, wcomb_ref, whht_ref, bn_ref, bhh_ref,
               g_ref, b_ref, wout_ref, bout_ref, wa_ref, wc_ref, bm1_ref,
               h2_ref, lg_ref, a_ref, c_ref):
    h = h_ref[...]
    r = r0_ref[...] + r1_ref[...]
    gi = jnp.dot(r, wcomb_ref[...], preferred_element_type=jnp.float32) + bn_ref[...]
    gh = jnp.dot(h, whht_ref[...], preferred_element_type=jnp.float32) + bhh_ref[...]
    i_r = gi[:, :H]; i_z = gi[:, H:2 * H]; i_n = gi[:, 2 * H:]
    h_r = gh[:, :H]; h_z = gh[:, H:2 * H]; h_n = gh[:, 2 * H:]
    rr = jax.nn.sigmoid(i_r + h_r)
    z = jax.nn.sigmoid(i_z + h_z)
    n = jnp.tanh(i_n + rr * h_n)
    hn = (1.0 - z) * n + z * h
    mu = jnp.mean(hn, axis=-1, keepdims=True)
    var = jnp.mean((hn - mu) ** 2, axis=-1, keepdims=True)
    h2 = (hn - mu) / jnp.sqrt(var + 1e-5) * g_ref[...] + b_ref[...]
    h2_ref[...] = h2
    lg_ref[...] = jnp.dot(h2, wout_ref[...], preferred_element_type=jnp.float32) + bout_ref[...]
    a_ref[...] = jnp.dot(h2, wa_ref[...], preferred_element_type=jnp.float32) + bm1_ref[...]
    c_ref[...] = jnp.dot(h2, wc_ref[...], preferred_element_type=jnp.float32)


def _step(r0, r1, h, wcomb, whht, bias_node, bhh, g, b, wout16, bout16, wa, wc, bm1):
    full = lambda i: (0, 0)
    rows = lambda i: (i, 0)
    return pl.pallas_call(
        _step_body,
        grid=(GRID,),
        in_specs=[
            pl.BlockSpec((BLK, H), rows),
            pl.BlockSpec((BLK, H), rows),
            pl.BlockSpec((BLK, H), rows),
            pl.BlockSpec((H, 3 * H), full),
            pl.BlockSpec((H, 3 * H), full),
            pl.BlockSpec((BLK, 3 * H), rows),
            pl.BlockSpec((1, 3 * H), full),
            pl.BlockSpec((1, H), full),
            pl.BlockSpec((1, H), full),
            pl.BlockSpec((H, 16), full),
            pl.BlockSpec((1, 16), full),
            pl.BlockSpec((H, H), full),
            pl.BlockSpec((H, H), full),
            pl.BlockSpec((1, H), full),
        ],
        out_specs=[
            pl.BlockSpec((BLK, H), rows),
            pl.BlockSpec((BLK, 16), rows),
            pl.BlockSpec((BLK, H), rows),
            pl.BlockSpec((BLK, H), rows),
        ],
        out_shape=[
            jax.ShapeDtypeStruct((N, H), jnp.float32),
            jax.ShapeDtypeStruct((N, 16), jnp.float32),
            jax.ShapeDtypeStruct((N, H), jnp.float32),
            jax.ShapeDtypeStruct((N, H), jnp.float32),
        ],
    )(r0, r1, h, wcomb, whht, bias_node, bhh, g, b, wout16, bout16, wa, wc, bm1)


def kernel(x, edge_index, W_in, b_in, ln_in_g, ln_in_b, pos, W_m1, b_m1,
           W_m2, b_m2, W_ih, b_ih, W_hh, b_hh, ln_g, ln_b, W_out, b_out):
    src = edge_index[0].astype(jnp.int32)
    dst = edge_index[1].astype(jnp.int32)

    x16 = jnp.pad(x.reshape(N, 10), ((0, 0), (0, 6)))
    win16 = jnp.pad(W_in, ((0, 6), (0, 0)))
    posf = jnp.tile(pos, (B, 1))
    wa = W_m1[:H]
    wc = W_m1[H:]
    wiht = W_ih.T
    whht = W_hh.T
    wout16 = jnp.pad(W_out, ((0, 0), (0, 7)))
    bout16 = jnp.pad(b_out, (0, 7)).reshape(1, 16)
    r1 = lambda v: v.reshape(1, -1)

    wcomb, brow = _wcomb(W_m2, wiht, b_m2.reshape(1, H))
    h0, a0, c0 = _init(x16, win16, r1(b_in), r1(ln_in_g), r1(ln_in_b), posf,
                       wa, wc, r1(b_m1))

    sc_edge = _make_sc_edge()
    cnt = _make_sc_cnt()(dst)
    bias_node = _bias(cnt[0, :N], cnt[1, :N], brow, r1(b_ih))
    bhh = r1(b_hh)
    g = r1(ln_g)
    b = r1(ln_b)

    def step(carry, _):
        h, a, c = carry
        r = sc_edge(a, c, src, dst)
        h2, lg, a2, c2 = _step(r[0, :N], r[1, :N], h, wcomb, whht, bias_node,
                               bhh, g, b, wout16, bout16, wa, wc, r1(b_m1))
        return (h2, a2, c2), lg

    (_, _, _), lgs = lax.scan(step, (h0, a0, c0), None, length=STEPS)
    return lgs[:, :, :9].reshape(STEPS, B, 81, 9)
